# baseline (device time: 89680 ns/iter reference)
import jax
import jax.numpy as jnp
from jax import lax
from jax.experimental import pallas as pl
from jax.experimental.pallas import tpu as pltpu

N_DEV = 4


def _ring_allgather(x, d2):
    m, n = x.shape
    dm, dn = d2.shape

    def body(x_ref, d_ref, xout_ref, dout_ref,
             xcomm, dcomm, xsend, xrecv, dsend, drecv):
        my = lax.axis_index("i")
        left = lax.rem(my + N_DEV - 1, N_DEV)
        right = lax.rem(my + 1, N_DEV)

        barrier = pltpu.get_barrier_semaphore()
        for nbr in (left, right):
            pl.semaphore_signal(
                barrier, inc=1,
                device_id=(nbr,), device_id_type=pl.DeviceIdType.MESH,
            )
        pl.semaphore_wait(barrier, 2)

        xout_ref[pl.ds(my * m, m), :] = x_ref[:, :]
        dout_ref[pl.ds(my * dm, dm), :] = d_ref[:, :]
        xcomm[0, :, :] = x_ref[:, :]
        dcomm[0, :, :] = d_ref[:, :]

        for h in range(N_DEV - 1):
            send_slot = h % 2
            recv_slot = (h + 1) % 2
            xr = pltpu.make_async_remote_copy(
                src_ref=xcomm.at[send_slot],
                dst_ref=xcomm.at[recv_slot],
                send_sem=xsend.at[send_slot],
                recv_sem=xrecv.at[recv_slot],
                device_id=(right,),
                device_id_type=pl.DeviceIdType.MESH,
            )
            dr = pltpu.make_async_remote_copy(
                src_ref=dcomm.at[send_slot],
                dst_ref=dcomm.at[recv_slot],
                send_sem=dsend.at[send_slot],
                recv_sem=drecv.at[recv_slot],
                device_id=(right,),
                device_id_type=pl.DeviceIdType.MESH,
            )
            xr.start()
            dr.start()
            xr.wait()
            dr.wait()

            origin = lax.rem(my + (2 * N_DEV - 1 - h), N_DEV)
            xout_ref[pl.ds(origin * m, m), :] = xcomm[recv_slot, :, :]
            dout_ref[pl.ds(origin * dm, dm), :] = dcomm[recv_slot, :, :]

    return pl.pallas_call(
        body,
        out_shape=(
            jax.ShapeDtypeStruct((N_DEV * m, n), x.dtype),
            jax.ShapeDtypeStruct((N_DEV * dm, dn), d2.dtype),
        ),
        in_specs=[
            pl.BlockSpec(memory_space=pltpu.VMEM),
            pl.BlockSpec(memory_space=pltpu.VMEM),
        ],
        out_specs=(
            pl.BlockSpec(memory_space=pltpu.VMEM),
            pl.BlockSpec(memory_space=pltpu.VMEM),
        ),
        scratch_shapes=[
            pltpu.VMEM((2, m, n), x.dtype),
            pltpu.VMEM((2, dm, dn), d2.dtype),
            pltpu.SemaphoreType.DMA((2,)),
            pltpu.SemaphoreType.DMA((2,)),
            pltpu.SemaphoreType.DMA((2,)),
            pltpu.SemaphoreType.DMA((2,)),
        ],
        compiler_params=pltpu.CompilerParams(collective_id=0),
    )(x, d2)


def kernel(x, dest):
    m = x.shape[0]
    d2 = dest.reshape(8, 128)
    x_all, d_all = _ring_allgather(x, d2)
    dest_all = d_all.reshape(N_DEV * m)

    order = jnp.argsort(dest_all, stable=True)
    my = lax.axis_index("i")
    my_order = lax.dynamic_slice(order, (my * m,), (m,))
    return jnp.take(x_all, my_order, axis=0)
